# Initial kernel scaffold; baseline (speedup 1.0000x reference)
import jax
import jax.numpy as jnp
from jax.experimental import pallas as pl


def kernel(x, edge_index, edge_weight, W1, b1, W2, b2):
    raise NotImplementedError("write your pallas kernel here")


# speedup vs baseline: 12.1701x; 12.1701x over previous
import functools

import jax
import jax.numpy as jnp
from jax import lax
from jax.experimental import pallas as pl
from jax.experimental.pallas import tpu as pltpu
from jax.experimental.pallas import tpu_sc as plsc

NC = 2
NS = 16
L = 16
NW = NC * NS
CHUNK = 128


def _sc_mesh():
    return plsc.VectorSubcoreMesh(core_axis_name="c", subcore_axis_name="s")


def _zero_vec_loop(ref, nwords):
    z = jnp.zeros((L,), jnp.float32)

    def body(i, carry):
        ref[pl.ds(i * L, L)] = z
        return carry

    lax.fori_loop(0, nwords // L, body, 0)


def _make_scalar_seg_kernel(n_pad, e_pad, n_vals, gather):
    ew_per_worker = e_pad // NW
    nch = ew_per_worker // CHUNK
    zb = n_pad // NS

    scratch = [
        pltpu.VMEM((1, CHUNK), jnp.int32),
        pltpu.VMEM((CHUNK,), jnp.float32),
        pltpu.VMEM((CHUNK,), jnp.float32),
        pltpu.VMEM((zb,), jnp.float32),
        pltpu.VMEM_SHARED((n_pad,), jnp.float32),
    ]
    if gather:
        scratch.insert(0, pltpu.VMEM((CHUNK,), jnp.int32))
        scratch.insert(0, pltpu.VMEM((CHUNK,), jnp.float32))

    @functools.partial(
        pl.kernel,
        out_type=jax.ShapeDtypeStruct((NC, n_pad), jnp.float32),
        mesh=_sc_mesh(),
        scratch_types=scratch,
    )
    def seg_kernel(*refs):
        if gather:
            (vals_hbm, src_hbm, dst_hbm, ew_hbm, out_hbm,
             gat_v, src_v, dst_v, ew_v, msg_v, zb_v, acc_sh) = refs
        else:
            (dst_hbm, ew_hbm, out_hbm,
             dst_v, ew_v, msg_v, zb_v, acc_sh) = refs
        c = lax.axis_index("c")
        s = lax.axis_index("s")
        wid = c * NS + s

        _zero_vec_loop(zb_v, zb)
        pltpu.sync_copy(zb_v, acc_sh.at[pl.ds(s * zb, zb)])
        plsc.subcore_barrier()

        def ebody(g, carry):
            off = wid * ew_per_worker + g * CHUNK
            pltpu.sync_copy(dst_hbm.at[pl.ds(off, CHUNK)], dst_v.at[0])
            pltpu.sync_copy(ew_hbm.at[pl.ds(off, CHUNK)], ew_v)
            if gather:
                pltpu.sync_copy(src_hbm.at[pl.ds(off, CHUNK)], src_v)
                pltpu.sync_copy(vals_hbm.at[src_v], gat_v)
                for j in range(CHUNK // L):
                    sl = pl.ds(j * L, L)
                    msg_v[sl] = ew_v[sl] * gat_v[sl]
                pltpu.sync_copy(msg_v, acc_sh.at[dst_v.at[0]], add=True)
            else:
                pltpu.sync_copy(ew_v, acc_sh.at[dst_v.at[0]], add=True)
            return carry

        lax.fori_loop(0, nch, ebody, 0)
        plsc.subcore_barrier()
        pltpu.sync_copy(acc_sh.at[pl.ds(s * zb, zb)],
                        out_hbm.at[c, pl.ds(s * zb, zb)])

    return seg_kernel


def _make_spmm_kernel(n_r, d, e_pad):
    ew_per_worker = e_pad // NW
    nch = ew_per_worker // CHUNK
    rows_per_tile = n_r // NS
    zrows = rows_per_tile
    while zrows * d * 4 > 65536:
        zrows //= 5
    ncopies = rows_per_tile // zrows

    @functools.partial(
        pl.kernel,
        out_type=jax.ShapeDtypeStruct((NC, n_r, d), jnp.float32),
        mesh=_sc_mesh(),
        scratch_types=[
            pltpu.VMEM((CHUNK,), jnp.int32),
            pltpu.VMEM((1, CHUNK), jnp.int32),
            pltpu.VMEM((CHUNK,), jnp.float32),
            pltpu.VMEM((CHUNK, d), jnp.float32),
            pltpu.VMEM((zrows, d), jnp.float32),
            pltpu.VMEM_SHARED((n_r, d), jnp.float32),
            pltpu.SemaphoreType.DMA,
        ],
    )
    def spmm_kernel(y_hbm, src_hbm, dst_hbm, ew_hbm, out_hbm,
                    src_v, dst_v, ew_v, rows_v, zb_v, acc_sh, sem):
        c = lax.axis_index("c")
        s = lax.axis_index("s")
        wid = c * NS + s

        z = jnp.zeros((L,), jnp.float32)

        def zbody(i, carry):
            for f in range(d // L):
                zb_v[i, pl.ds(f * L, L)] = z
            return carry

        lax.fori_loop(0, zrows, zbody, 0)
        for r in range(ncopies):
            pltpu.sync_copy(
                zb_v, acc_sh.at[pl.ds(s * rows_per_tile + r * zrows, zrows), :])
        plsc.subcore_barrier()

        def ebody(g, carry):
            off = wid * ew_per_worker + g * CHUNK
            pltpu.sync_copy(src_hbm.at[pl.ds(off, CHUNK)], src_v)
            pltpu.sync_copy(dst_hbm.at[pl.ds(off, CHUNK)], dst_v.at[0])
            pltpu.sync_copy(ew_hbm.at[pl.ds(off, CHUNK)], ew_v)
            pltpu.async_copy(y_hbm.at[src_v], rows_v, sem).wait()

            def sbody(j, icarry):
                ewv = ew_v[pl.ds(j * L, L)]
                base = j * L
                for lane in range(L):
                    sv = ewv[lane]
                    for f in range(d // L):
                        sl = pl.ds(f * L, L)
                        rows_v[base + lane, sl] = rows_v[base + lane, sl] * sv
                return icarry

            lax.fori_loop(0, CHUNK // L, sbody, 0)
            pltpu.sync_copy(rows_v, acc_sh.at[dst_v.at[0]], add=True)
            return carry

        lax.fori_loop(0, nch, ebody, 0)
        plsc.subcore_barrier()
        pltpu.sync_copy(
            acc_sh.at[pl.ds(s * rows_per_tile, rows_per_tile), :],
            out_hbm.at[c, pl.ds(s * rows_per_tile, rows_per_tile), :])

    return spmm_kernel


def _tc1_body(x_ref, w_ref, dpt_ref, y_ref, dis_ref):
    xw = jnp.dot(x_ref[...], w_ref[...], preferred_element_type=jnp.float32)
    deg = dpt_ref[:, 0] + dpt_ref[:, 1] + 1.0
    dis = jnp.where(deg > 0,
                    lax.rsqrt(jnp.maximum(deg, 1e-12)),
                    0.0)[:, None]
    y_ref[...] = xw * dis
    dis_ref[...] = dis


def _tc2_body(p_ref, y_ref, dis_ref, w2_ref, b1_ref, y2_ref):
    agg = p_ref[0] + p_ref[1] + y_ref[...]
    dis = dis_ref[...]
    h = jnp.maximum(dis * agg + b1_ref[...], 0.0)
    h2 = jnp.dot(h, w2_ref[...], preferred_element_type=jnp.float32)
    y2_ref[...] = dis * h2


def _tc3_body(qt_ref, y2_ref, dis_ref, b2_ref, o_ref):
    agg = (qt_ref[:, 0] + qt_ref[:, 1])[:, None] + y2_ref[...]
    o_ref[...] = dis_ref[...] * agg + b2_ref[0, 0]


def kernel(x, edge_index, edge_weight, W1, b1, W2, b2):
    n, d_in = x.shape
    d_hid = W1.shape[1]
    d_out = W2.shape[1]
    e = edge_weight.shape[0]

    epw = CHUNK * -(-e // (NW * CHUNK))
    e_pad = NW * epw
    src = edge_index[0].astype(jnp.int32)
    dst = edge_index[1].astype(jnp.int32)
    ew = edge_weight.astype(jnp.float32)
    pad = e_pad - e
    if pad:
        src = jnp.concatenate([src, jnp.zeros((pad,), jnp.int32)])
        dst = jnp.concatenate([dst, jnp.zeros((pad,), jnp.int32)])
        ew = jnp.concatenate([ew, jnp.zeros((pad,), jnp.float32)])

    n_pad = NS * L * -(-n // (NS * L))

    rb = 1000
    grid = n // rb

    deg_kernel = _make_scalar_seg_kernel(n_pad, e_pad, n, gather=False)
    dp = deg_kernel(dst, ew)
    dpt = dp.T[:n]

    y1, dis = pl.pallas_call(
        _tc1_body,
        grid=(grid,),
        in_specs=[
            pl.BlockSpec((rb, d_in), lambda i: (i, 0)),
            pl.BlockSpec((d_in, d_hid), lambda i: (0, 0)),
            pl.BlockSpec((rb, 2), lambda i: (i, 0)),
        ],
        out_specs=[
            pl.BlockSpec((rb, d_hid), lambda i: (i, 0)),
            pl.BlockSpec((rb, 1), lambda i: (i, 0)),
        ],
        out_shape=[
            jax.ShapeDtypeStruct((n, d_hid), jnp.float32),
            jax.ShapeDtypeStruct((n, 1), jnp.float32),
        ],
    )(x, W1, dpt)

    n_r = NS * 8 * -(-n // (NS * 8))
    spmm = _make_spmm_kernel(n_r, d_hid, e_pad)
    p = spmm(y1, src, dst, ew)

    y2 = pl.pallas_call(
        _tc2_body,
        grid=(grid,),
        in_specs=[
            pl.BlockSpec((NC, rb, d_hid), lambda i: (0, i, 0)),
            pl.BlockSpec((rb, d_hid), lambda i: (i, 0)),
            pl.BlockSpec((rb, 1), lambda i: (i, 0)),
            pl.BlockSpec((d_hid, d_out), lambda i: (0, 0)),
            pl.BlockSpec((1, d_hid), lambda i: (0, 0)),
        ],
        out_specs=pl.BlockSpec((rb, d_out), lambda i: (i, 0)),
        out_shape=jax.ShapeDtypeStruct((n, d_out), jnp.float32),
    )(p, y1, dis, W2, b1.reshape(1, d_hid))

    seg = _make_scalar_seg_kernel(n_pad, e_pad, n, gather=True)
    q = seg(y2.reshape(n), src, dst, ew)
    qt = q.T[:n]

    out = pl.pallas_call(
        _tc3_body,
        grid=(grid,),
        in_specs=[
            pl.BlockSpec((rb, 2), lambda i: (i, 0)),
            pl.BlockSpec((rb, 1), lambda i: (i, 0)),
            pl.BlockSpec((rb, 1), lambda i: (i, 0)),
            pl.BlockSpec((1, 1), lambda i: (0, 0)),
        ],
        out_specs=pl.BlockSpec((rb, 1), lambda i: (i, 0)),
        out_shape=jax.ShapeDtypeStruct((n, d_out), jnp.float32),
    )(qt, y2, dis, b2.reshape(1, 1))

    return out


# speedup vs baseline: 15.1360x; 1.2437x over previous
import functools

import jax
import jax.numpy as jnp
from jax import lax
from jax.experimental import pallas as pl
from jax.experimental.pallas import tpu as pltpu
from jax.experimental.pallas import tpu_sc as plsc

NC = 2
NS = 16
L = 16
NW = NC * NS
CHUNK = 128
WIN = 8
MB = 4


def _sc_mesh():
    return plsc.VectorSubcoreMesh(core_axis_name="c", subcore_axis_name="s")


def _zero_vec_loop(ref, nwords):
    z = jnp.zeros((L,), jnp.float32)

    def body(i, carry):
        ref[pl.ds(i * L, L)] = z
        return carry

    lax.fori_loop(0, nwords // L, body, 0)


def _make_scalar_seg_kernel(n_pad, e_pad, gather):
    epw = e_pad // NW
    nch = epw // CHUNK
    zb = n_pad // NS

    scratch = [
        pltpu.VMEM((nch, CHUNK), jnp.int32),
        pltpu.VMEM((epw,), jnp.float32),
        pltpu.VMEM((zb,), jnp.float32),
        pltpu.VMEM_SHARED((n_pad,), jnp.float32),
        pltpu.SemaphoreType.DMA,
    ]
    if gather:
        scratch.insert(0, pltpu.VMEM((epw,), jnp.int32))
        scratch.insert(1, pltpu.VMEM((epw,), jnp.float32))
        scratch.insert(2, pltpu.VMEM((epw,), jnp.float32))

    @functools.partial(
        pl.kernel,
        out_type=jax.ShapeDtypeStruct((NC, n_pad), jnp.float32),
        mesh=_sc_mesh(),
        scratch_types=scratch,
    )
    def seg_kernel(*refs):
        if gather:
            (vals_hbm, src_hbm, dst_hbm, ew_hbm, out_hbm,
             src_v, gat_v, msg_v, dst_v, ew_v, zb_v, acc_sh, sem) = refs
        else:
            (dst_hbm, ew_hbm, out_hbm,
             dst_v, ew_v, zb_v, acc_sh, sem) = refs
        c = lax.axis_index("c")
        s = lax.axis_index("s")
        wid = c * NS + s

        _zero_vec_loop(zb_v, zb)
        pltpu.sync_copy(ew_hbm.at[pl.ds(wid * epw, epw)], ew_v)
        pltpu.sync_copy(dst_hbm.at[pl.ds(wid * nch, nch), :], dst_v)
        if gather:
            pltpu.sync_copy(src_hbm.at[pl.ds(wid * epw, epw)], src_v)
        pltpu.sync_copy(zb_v, acc_sh.at[pl.ds(s * zb, zb)])
        plsc.subcore_barrier()

        if gather:
            def gbody(g, carry):
                sl = pl.ds(g * CHUNK, CHUNK)
                pltpu.async_copy(vals_hbm.at[src_v.at[sl]], gat_v.at[sl], sem)

                @pl.when(g >= WIN)
                def _():
                    dsl = pl.ds((g - WIN) * CHUNK, CHUNK)
                    pltpu.make_async_copy(
                        vals_hbm.at[src_v.at[dsl]], gat_v.at[dsl], sem).wait()
                return carry

            lax.fori_loop(0, nch, gbody, 0)

            def gdrain(k, carry):
                dsl = pl.ds((nch - WIN + k) * CHUNK, CHUNK)
                pltpu.make_async_copy(
                    vals_hbm.at[src_v.at[dsl]], gat_v.at[dsl], sem).wait()
                return carry

            lax.fori_loop(0, WIN, gdrain, 0)

            def mbody(i, carry):
                sl = pl.ds(i * L, L)
                msg_v[sl] = ew_v[sl] * gat_v[sl]
                return carry

            lax.fori_loop(0, epw // L, mbody, 0)
            data_v = msg_v
        else:
            data_v = ew_v

        def sbody(g, carry):
            sl = pl.ds(g * CHUNK, CHUNK)
            pltpu.async_copy(data_v.at[sl], acc_sh.at[dst_v.at[g]], sem,
                             add=True)

            @pl.when(g >= WIN)
            def _():
                dsl = pl.ds((g - WIN) * CHUNK, CHUNK)
                pltpu.make_async_copy(
                    data_v.at[dsl], acc_sh.at[dst_v.at[g - WIN]], sem).wait()
            return carry

        lax.fori_loop(0, nch, sbody, 0)

        def sdrain(k, carry):
            g = nch - WIN + k
            dsl = pl.ds(g * CHUNK, CHUNK)
            pltpu.make_async_copy(
                data_v.at[dsl], acc_sh.at[dst_v.at[g]], sem).wait()
            return carry

        lax.fori_loop(0, WIN, sdrain, 0)
        plsc.subcore_barrier()
        pltpu.sync_copy(acc_sh.at[pl.ds(s * zb, zb)],
                        out_hbm.at[c, pl.ds(s * zb, zb)])

    return seg_kernel


def _make_spmm_kernel(n_r, d, e_pad):
    epw = e_pad // NW
    nch = epw // CHUNK
    assert nch % MB == 0
    ncb = nch // MB
    assert ncb % 2 == 0
    epb = ncb * CHUNK
    rows_per_tile = n_r // NS
    ncopies = rows_per_tile // CHUNK

    @functools.partial(
        pl.kernel,
        out_type=jax.ShapeDtypeStruct((NC, n_r, d), jnp.float32),
        mesh=_sc_mesh(),
        scratch_types=[
            pltpu.VMEM((epb,), jnp.int32),
            pltpu.VMEM((ncb, CHUNK), jnp.int32),
            pltpu.VMEM((epb,), jnp.float32),
            [pltpu.VMEM((CHUNK, d), jnp.float32) for _ in range(2)],
            pltpu.VMEM_SHARED((n_r, d), jnp.float32),
            [pltpu.SemaphoreType.DMA for _ in range(2)],
            [pltpu.SemaphoreType.DMA for _ in range(2)],
        ],
    )
    def spmm_kernel(y_hbm, src_hbm, dst_hbm, ew_hbm, out_hbm,
                    src_v, dst_v, ew_v, rows, acc_sh, gsem, ssem):
        c = lax.axis_index("c")
        s = lax.axis_index("s")
        wid = c * NS + s

        z = jnp.zeros((L,), jnp.float32)

        def zbody(i, carry):
            for f in range(d // L):
                rows[0][i, pl.ds(f * L, L)] = z
            return carry

        lax.fori_loop(0, CHUNK, zbody, 0)
        for r in range(ncopies):
            pltpu.sync_copy(
                rows[0],
                acc_sh.at[pl.ds(s * rows_per_tile + r * CHUNK, CHUNK), :])
        plsc.subcore_barrier()

        def fire_gather(k, b):
            pltpu.async_copy(
                y_hbm.at[src_v.at[pl.ds(k * CHUNK, CHUNK)]], rows[b], gsem[b])

        def wait_gather(k, b):
            pltpu.make_async_copy(
                y_hbm.at[src_v.at[pl.ds(k * CHUNK, CHUNK)]], rows[b],
                gsem[b]).wait()

        def fire_scatter(k, b):
            pltpu.async_copy(rows[b], acc_sh.at[dst_v.at[k]], ssem[b],
                             add=True)

        def wait_scatter(k, b):
            pltpu.make_async_copy(rows[b], acc_sh.at[dst_v.at[k]],
                                  ssem[b]).wait()

        def mbody(m, carry):
            boff = wid * epw + m * epb
            pltpu.sync_copy(src_hbm.at[pl.ds(boff, epb)], src_v)
            pltpu.sync_copy(dst_hbm.at[wid * MB + m], dst_v)
            pltpu.sync_copy(ew_hbm.at[pl.ds(boff, epb)], ew_v)

            fire_gather(0, 0)

            def kbody(k2, icarry):
                for r in range(2):
                    k = k2 * 2 + r

                    @pl.when(k >= 1)
                    def _():
                        wait_scatter(k - 1, 1 - r)

                    @pl.when(k + 1 < ncb)
                    def _():
                        fire_gather(k + 1, 1 - r)

                    wait_gather(k, r)

                    def sbody(j, jcarry):
                        ewv = ew_v[pl.ds(k * CHUNK + j * L, L)]
                        rv = rows[r]
                        base = j * L
                        for lane in range(L):
                            sv = ewv[lane]
                            for f in range(d // L):
                                sl = pl.ds(f * L, L)
                                rv[base + lane, sl] = rv[base + lane, sl] * sv
                        return jcarry

                    lax.fori_loop(0, CHUNK // L, sbody, 0)
                    fire_scatter(k, r)
                return icarry

            lax.fori_loop(0, ncb // 2, kbody, 0)
            wait_scatter(ncb - 1, 1)
            return carry

        lax.fori_loop(0, MB, mbody, 0)

        plsc.subcore_barrier()
        pltpu.sync_copy(
            acc_sh.at[pl.ds(s * rows_per_tile, rows_per_tile), :],
            out_hbm.at[c, pl.ds(s * rows_per_tile, rows_per_tile), :])

    return spmm_kernel


def _tc1_body(x_ref, w_ref, dpt_ref, y_ref, dis_ref):
    xw = jnp.dot(x_ref[...], w_ref[...], preferred_element_type=jnp.float32)
    deg = dpt_ref[:, 0] + dpt_ref[:, 1] + 1.0
    dis = jnp.where(deg > 0,
                    lax.rsqrt(jnp.maximum(deg, 1e-12)),
                    0.0)[:, None]
    y_ref[...] = xw * dis
    dis_ref[...] = dis


def _tc2_body(p_ref, y_ref, dis_ref, w2_ref, b1_ref, y2_ref):
    agg = p_ref[0] + p_ref[1] + y_ref[...]
    dis = dis_ref[...]
    h = jnp.maximum(dis * agg + b1_ref[...], 0.0)
    h2 = jnp.dot(h, w2_ref[...], preferred_element_type=jnp.float32)
    y2_ref[...] = dis * h2


def _tc3_body(qt_ref, y2_ref, dis_ref, b2_ref, o_ref):
    agg = (qt_ref[:, 0] + qt_ref[:, 1])[:, None] + y2_ref[...]
    o_ref[...] = dis_ref[...] * agg + b2_ref[0, 0]


def kernel(x, edge_index, edge_weight, W1, b1, W2, b2):
    n, d_in = x.shape
    d_hid = W1.shape[1]
    d_out = W2.shape[1]
    e = edge_weight.shape[0]

    nch = 2 * MB * -(-e // (NW * CHUNK * 2 * MB))
    epw = nch * CHUNK
    e_pad = NW * epw
    src = edge_index[0].astype(jnp.int32)
    dst = edge_index[1].astype(jnp.int32)
    ew = edge_weight.astype(jnp.float32)
    pad = e_pad - e
    if pad:
        src = jnp.concatenate([src, jnp.zeros((pad,), jnp.int32)])
        dst = jnp.concatenate([dst, jnp.zeros((pad,), jnp.int32)])
        ew = jnp.concatenate([ew, jnp.zeros((pad,), jnp.float32)])
    dst2d = dst.reshape(NW * nch, CHUNK)

    n_pad = NS * L * -(-n // (NS * L))

    rb = 1000
    grid = n // rb

    deg_kernel = _make_scalar_seg_kernel(n_pad, e_pad, gather=False)
    dp = deg_kernel(dst2d, ew)
    dpt = dp.T[:n]

    y1, dis = pl.pallas_call(
        _tc1_body,
        grid=(grid,),
        in_specs=[
            pl.BlockSpec((rb, d_in), lambda i: (i, 0)),
            pl.BlockSpec((d_in, d_hid), lambda i: (0, 0)),
            pl.BlockSpec((rb, 2), lambda i: (i, 0)),
        ],
        out_specs=[
            pl.BlockSpec((rb, d_hid), lambda i: (i, 0)),
            pl.BlockSpec((rb, 1), lambda i: (i, 0)),
        ],
        out_shape=[
            jax.ShapeDtypeStruct((n, d_hid), jnp.float32),
            jax.ShapeDtypeStruct((n, 1), jnp.float32),
        ],
    )(x, W1, dpt)

    n_r = NS * 8 * -(-n // (NS * 8))
    spmm = _make_spmm_kernel(n_r, d_hid, e_pad)
    dst3d = dst.reshape(NW * MB, nch // MB, CHUNK)
    p = spmm(y1, src, dst3d, ew)

    y2 = pl.pallas_call(
        _tc2_body,
        grid=(grid,),
        in_specs=[
            pl.BlockSpec((NC, rb, d_hid), lambda i: (0, i, 0)),
            pl.BlockSpec((rb, d_hid), lambda i: (i, 0)),
            pl.BlockSpec((rb, 1), lambda i: (i, 0)),
            pl.BlockSpec((d_hid, d_out), lambda i: (0, 0)),
            pl.BlockSpec((1, d_hid), lambda i: (0, 0)),
        ],
        out_specs=pl.BlockSpec((rb, d_out), lambda i: (i, 0)),
        out_shape=jax.ShapeDtypeStruct((n, d_out), jnp.float32),
    )(p, y1, dis, W2, b1.reshape(1, d_hid))

    seg = _make_scalar_seg_kernel(n_pad, e_pad, gather=True)
    q = seg(y2.reshape(n), src, dst2d, ew)
    qt = q.T[:n]

    out = pl.pallas_call(
        _tc3_body,
        grid=(grid,),
        in_specs=[
            pl.BlockSpec((rb, 2), lambda i: (i, 0)),
            pl.BlockSpec((rb, 1), lambda i: (i, 0)),
            pl.BlockSpec((rb, 1), lambda i: (i, 0)),
            pl.BlockSpec((1, 1), lambda i: (0, 0)),
        ],
        out_specs=pl.BlockSpec((rb, 1), lambda i: (i, 0)),
        out_shape=jax.ShapeDtypeStruct((n, d_out), jnp.float32),
    )(qt, y2, dis, b2.reshape(1, 1))

    return out
